# baseline (speedup 1.0000x reference)
import functools

import numpy as np
import jax
import jax.numpy as jnp
from jax import lax
from jax.experimental import pallas as pl
from jax.experimental.pallas import tpu as pltpu
from jax.experimental.pallas import tpu_sc as plsc

_Z = 32
_NT = _Z * (_Z + 1) // 2
_B = 4096
_NE = 64
_NC = 1000


def _build_scatter_matrix():
    s = np.zeros((_NT, _Z * _Z), dtype=np.float32)
    t = 0
    for i in range(_Z):
        for j in range(i + 1):
            s[t, i * _Z + j] = 1.0
            t += 1
    return s


_SCATTER_NP = _build_scatter_matrix()


def _tc_transpose_tables(cov_sp_t2d, mu_sp_t2d):
    def body(in_c_ref, in_m_ref, out_c_ref, out_m_ref):
        tr = jnp.transpose(in_c_ref[...], (1, 0))
        u = lax.bitcast_convert_type(tr.astype(jnp.bfloat16),
                                     jnp.uint16).astype(jnp.uint32)
        word = u[:, :_NT // 2] | (u[:, _NT // 2:] << 16)
        out_c_ref[...] = lax.bitcast_convert_type(word, jnp.float32)
        out_m_ref[...] = jnp.transpose(in_m_ref[...], (1, 0))

    return pl.pallas_call(
        body,
        grid=(_NE,),
        in_specs=[
            pl.BlockSpec((_NT, _NC), lambda e: (e, 0)),
            pl.BlockSpec((_Z, _NC), lambda e: (e, 0)),
        ],
        out_specs=[
            pl.BlockSpec((_NC, _NT // 2), lambda e: (e, 0)),
            pl.BlockSpec((_NC, _Z), lambda e: (e, 0)),
        ],
        out_shape=[
            jax.ShapeDtypeStruct((_NE * _NC, _NT // 2), jnp.float32),
            jax.ShapeDtypeStruct((_NE * _NC, _Z), jnp.float32),
        ],
    )(cov_sp_t2d, mu_sp_t2d)


def _sc_gather_mu(y_flat, e_flat, mu_table):
    info = plsc.get_sparse_core_info()
    num_cores, num_subcores = info.num_cores, info.num_subcores
    nw = num_cores * num_subcores
    bpw = _B // nw
    lanes = info.num_lanes

    mesh = plsc.VectorSubcoreMesh(core_axis_name="c", subcore_axis_name="s")

    @functools.partial(
        pl.kernel,
        out_type=jax.ShapeDtypeStruct((_B, _Z), jnp.float32),
        mesh=mesh,
        scratch_types=[
            pltpu.VMEM((bpw,), jnp.int32),
            pltpu.VMEM((bpw,), jnp.int32),
            pltpu.VMEM((bpw,), jnp.int32),
            pltpu.VMEM((bpw, _Z), jnp.float32),
            pltpu.SemaphoreType.DMA,
        ],
        compiler_params=pltpu.CompilerParams(use_tc_tiling_on_sc=False),
    )
    def gather_mu_kernel(y_hbm, e_hbm, mu_hbm, mu_out,
                         y_v, e_v, idx_v, mu_rows, sem):
        wid = lax.axis_index("s") * num_cores + lax.axis_index("c")
        base = wid * bpw
        pltpu.sync_copy(y_hbm.at[pl.ds(base, bpw)], y_v)
        pltpu.sync_copy(e_hbm.at[pl.ds(base, bpw)], e_v)
        for i in range(bpw // lanes):
            sl = pl.ds(i * lanes, lanes)
            idx_v[sl] = e_v[sl] * _NC + y_v[sl]
        pltpu.async_copy(mu_hbm.at[idx_v], mu_rows, sem).wait()
        pltpu.sync_copy(mu_rows, mu_out.at[pl.ds(base, bpw)])

    return gather_mu_kernel(y_flat, e_flat, mu_table)


def _tc_assemble(e_row, muc_t, cov_causal, y_flat, e_flat, mu_s_rows,
                 cov_rm, scatter, interpret=False):
    bb = 256
    grid = _B // bb

    def body(e_ref, muct_ref, covc_ref, y_sref, e_sref, mus_ref, tab_ref,
             s_ref, mu_out_ref, cov_out_ref, cc_tab_ref, rows_buf, sem0, sem1):
        step = pl.program_id(0)

        def fire(block, slot, sem):
            base = block * bb

            def one(r, carry):
                row = e_sref[base + r] * _NC + y_sref[base + r]
                pltpu.make_async_copy(
                    tab_ref.at[pl.ds(row, 1)],
                    rows_buf.at[pl.ds(slot * bb + r, 1)], sem).start()
                return carry

            lax.fori_loop(0, bb, one, 0)

        def drain(slot, sem):
            pltpu.make_async_copy(tab_ref.at[pl.ds(0, bb)],
                                  rows_buf.at[pl.ds(slot * bb, bb)],
                                  sem).wait()

        @pl.when(step == 0)
        def _():
            fire(0, 0, sem0)

        nxt = step + 1

        @pl.when(jnp.logical_and(nxt < grid, nxt % 2 == 0))
        def _():
            fire(nxt, 0, sem0)

        @pl.when(jnp.logical_and(nxt < grid, nxt % 2 == 1))
        def _():
            fire(nxt, 1, sem1)

        @pl.when(step % 2 == 0)
        def _():
            drain(0, sem0)

        @pl.when(step % 2 == 1)
        def _():
            drain(1, sem1)

        @pl.when(step == 0)
        def _():
            lc = jnp.reshape(
                jnp.dot(covc_ref[...], s_ref[...],
                        preferred_element_type=jnp.float32),
                (_NE, _Z, _Z))
            cc = lax.dot_general(lc, lc, (((2,), (2,)), ((0,), (0,))),
                                 preferred_element_type=jnp.float32)
            cc_tab_ref[...] = jnp.transpose(
                jnp.reshape(cc, (_NE, _Z * _Z)), (1, 0))

        onehot_t = (lax.broadcast_in_dim(e_ref[...], (_NE, bb), (0, 1))
                    == lax.broadcasted_iota(jnp.int32, (_NE, bb), 0)
                    ).astype(jnp.float32)
        mu_out_ref[0:_Z, :] = jnp.dot(muct_ref[...], onehot_t,
                                      preferred_element_type=jnp.float32)
        mu_out_ref[_Z:2 * _Z, :] = jnp.transpose(mus_ref[...], (1, 0))

        cov_c_t = jnp.dot(cc_tab_ref[...], onehot_t,
                          preferred_element_type=jnp.float32)
        w = lax.bitcast_convert_type(rows_buf[pl.ds((step % 2) * bb, bb)],
                                     jnp.uint32)
        lo = lax.bitcast_convert_type((w & 0xFFFF).astype(jnp.uint16),
                                      jnp.bfloat16)
        hi = lax.bitcast_convert_type((w >> 16).astype(jnp.uint16),
                                      jnp.bfloat16)
        covs_rows = jnp.concatenate([lo, hi], axis=1)
        ls = jnp.reshape(
            jnp.dot(covs_rows, s_ref[...].astype(jnp.bfloat16),
                    preferred_element_type=jnp.float32),
            (bb, _Z, _Z))
        cov_s = lax.dot_general(ls, ls, (((2,), (2,)), ((0,), (0,))),
                                preferred_element_type=jnp.float32)
        cov_s_t = jnp.transpose(jnp.reshape(cov_s, (bb, _Z * _Z)), (1, 0))
        zero = jnp.zeros((_Z, _Z, bb), jnp.float32)
        cov_out_ref[0:_Z, 0:_Z, :] = jnp.reshape(cov_c_t, (_Z, _Z, bb))
        cov_out_ref[0:_Z, _Z:2 * _Z, :] = zero
        cov_out_ref[_Z:2 * _Z, 0:_Z, :] = zero
        cov_out_ref[_Z:2 * _Z, _Z:2 * _Z, :] = jnp.reshape(cov_s_t,
                                                           (_Z, _Z, bb))

    return pl.pallas_call(
        body,
        grid=(grid,),
        in_specs=[
            pl.BlockSpec((1, bb), lambda i: (0, i)),
            pl.BlockSpec((_Z, _NE), lambda i: (0, 0)),
            pl.BlockSpec((_NE, _NT), lambda i: (0, 0)),
            pl.BlockSpec(memory_space=pltpu.SMEM),
            pl.BlockSpec(memory_space=pltpu.SMEM),
            pl.BlockSpec((bb, _Z), lambda i: (i, 0)),
            pl.BlockSpec(memory_space=pl.ANY),
            pl.BlockSpec((_NT, _Z * _Z), lambda i: (0, 0)),
        ],
        out_specs=[
            pl.BlockSpec((2 * _Z, bb), lambda i: (0, i)),
            pl.BlockSpec((2 * _Z, 2 * _Z, bb), lambda i: (0, 0, i)),
        ],
        out_shape=[
            jax.ShapeDtypeStruct((2 * _Z, _B), jnp.float32),
            jax.ShapeDtypeStruct((2 * _Z, 2 * _Z, _B), jnp.float32),
        ],
        scratch_shapes=[
            pltpu.VMEM((_Z * _Z, _NE), jnp.float32),
            pltpu.VMEM((2 * bb, _NT // 2), jnp.float32),
            pltpu.SemaphoreType.DMA,
            pltpu.SemaphoreType.DMA,
        ],
        interpret=interpret,
    )(e_row, muc_t, cov_causal, y_flat, e_flat, mu_s_rows, cov_rm, scatter)


def kernel(y, e, mu_causal, cov_causal, mu_spurious, cov_spurious):
    y_flat = y.reshape(_B).astype(jnp.int32)
    e_flat = e.reshape(_B).astype(jnp.int32)
    scatter = jnp.asarray(_SCATTER_NP)

    cov_sp_t = jnp.transpose(cov_spurious, (1, 2, 0)).reshape(_NE * _NT, _NC)
    mu_sp_t = jnp.transpose(mu_spurious, (1, 2, 0)).reshape(_NE * _Z, _NC)

    cov_rm, mu_rm = _tc_transpose_tables(cov_sp_t, mu_sp_t)
    mu_s_rows = _sc_gather_mu(y_flat, e_flat, mu_rm)

    mu_t, cov_t = _tc_assemble(e_flat.reshape(1, _B), mu_causal.T, cov_causal,
                               y_flat, e_flat, mu_s_rows, cov_rm, scatter)
    return (jnp.transpose(mu_t, (1, 0)), jnp.transpose(cov_t, (2, 0, 1)))


# speedup vs baseline: 1.0120x; 1.0120x over previous
import functools

import numpy as np
import jax
import jax.numpy as jnp
from jax import lax
from jax.experimental import pallas as pl
from jax.experimental.pallas import tpu as pltpu
from jax.experimental.pallas import tpu_sc as plsc

_Z = 32
_NT = _Z * (_Z + 1) // 2
_B = 4096
_NE = 64
_NC = 1000


def _build_scatter_matrix():
    s = np.zeros((_NT, _Z * _Z), dtype=np.float32)
    t = 0
    for i in range(_Z):
        for j in range(i + 1):
            s[t, i * _Z + j] = 1.0
            t += 1
    return s


_SCATTER_NP = _build_scatter_matrix()


def _tc_transpose_table(cov_sp_t2d):
    def body(in_c_ref, out_c_ref):
        tr = jnp.transpose(in_c_ref[...], (1, 0))
        u = lax.bitcast_convert_type(tr.astype(jnp.bfloat16),
                                     jnp.uint16).astype(jnp.uint32)
        word = u[:, :_NT // 2] | (u[:, _NT // 2:] << 16)
        out_c_ref[...] = lax.bitcast_convert_type(word, jnp.float32)

    return pl.pallas_call(
        body,
        grid=(_NE,),
        in_specs=[pl.BlockSpec((_NT, _NC), lambda e: (e, 0))],
        out_specs=pl.BlockSpec((_NC, _NT // 2), lambda e: (e, 0)),
        out_shape=jax.ShapeDtypeStruct((_NE * _NC, _NT // 2), jnp.float32),
    )(cov_sp_t2d)


def _sc_gather_mu(y_flat, e_flat, mu_table):
    info = plsc.get_sparse_core_info()
    num_cores, num_subcores = info.num_cores, info.num_subcores
    nw = num_cores * num_subcores
    bpw = _B // nw
    lanes = info.num_lanes

    mesh = plsc.VectorSubcoreMesh(core_axis_name="c", subcore_axis_name="s")

    @functools.partial(
        pl.kernel,
        out_type=jax.ShapeDtypeStruct((_B, _Z), jnp.float32),
        mesh=mesh,
        scratch_types=[
            pltpu.VMEM((bpw,), jnp.int32),
            pltpu.VMEM((bpw,), jnp.int32),
            pltpu.VMEM((bpw,), jnp.int32),
            pltpu.VMEM((bpw, _Z), jnp.float32),
            pltpu.SemaphoreType.DMA,
        ],
        compiler_params=pltpu.CompilerParams(use_tc_tiling_on_sc=False),
    )
    def gather_mu_kernel(y_hbm, e_hbm, mu_hbm, mu_out,
                         y_v, e_v, idx_v, mu_rows, sem):
        wid = lax.axis_index("s") * num_cores + lax.axis_index("c")
        base = wid * bpw
        pltpu.sync_copy(y_hbm.at[pl.ds(base, bpw)], y_v)
        pltpu.sync_copy(e_hbm.at[pl.ds(base, bpw)], e_v)
        for i in range(bpw // lanes):
            sl = pl.ds(i * lanes, lanes)
            idx_v[sl] = y_v[sl] * _NE + e_v[sl]
        pltpu.async_copy(mu_hbm.at[idx_v], mu_rows, sem).wait()
        pltpu.sync_copy(mu_rows, mu_out.at[pl.ds(base, bpw)])

    return gather_mu_kernel(y_flat, e_flat, mu_table)


def _tc_assemble(e_row, muc_t, cov_causal, y_flat, e_flat, mu_s_rows,
                 cov_rm, scatter, interpret=False):
    bb = 256
    grid = _B // bb

    def body(e_ref, muct_ref, covc_ref, y_sref, e_sref, mus_ref, tab_ref,
             s_ref, mu_out_ref, cov_out_ref, cc_tab_ref, rows_buf, sem0, sem1):
        step = pl.program_id(0)

        def fire(block, slot, sem):
            base = block * bb

            def one(r, carry):
                row = e_sref[base + r] * _NC + y_sref[base + r]
                pltpu.make_async_copy(
                    tab_ref.at[pl.ds(row, 1)],
                    rows_buf.at[pl.ds(slot * bb + r, 1)], sem).start()
                return carry

            lax.fori_loop(0, bb, one, 0)

        def drain(slot, sem):
            pltpu.make_async_copy(tab_ref.at[pl.ds(0, bb)],
                                  rows_buf.at[pl.ds(slot * bb, bb)],
                                  sem).wait()

        @pl.when(step == 0)
        def _():
            fire(0, 0, sem0)

        nxt = step + 1

        @pl.when(jnp.logical_and(nxt < grid, nxt % 2 == 0))
        def _():
            fire(nxt, 0, sem0)

        @pl.when(jnp.logical_and(nxt < grid, nxt % 2 == 1))
        def _():
            fire(nxt, 1, sem1)

        @pl.when(step % 2 == 0)
        def _():
            drain(0, sem0)

        @pl.when(step % 2 == 1)
        def _():
            drain(1, sem1)

        @pl.when(step == 0)
        def _():
            lc = jnp.reshape(
                jnp.dot(covc_ref[...], s_ref[...],
                        preferred_element_type=jnp.float32),
                (_NE, _Z, _Z))
            cc = lax.dot_general(lc, lc, (((2,), (2,)), ((0,), (0,))),
                                 preferred_element_type=jnp.float32)
            cc_tab_ref[...] = jnp.transpose(
                jnp.reshape(cc, (_NE, _Z * _Z)), (1, 0))

        onehot_t = (lax.broadcast_in_dim(e_ref[...], (_NE, bb), (0, 1))
                    == lax.broadcasted_iota(jnp.int32, (_NE, bb), 0)
                    ).astype(jnp.float32)
        mu_out_ref[0:_Z, :] = jnp.dot(muct_ref[...], onehot_t,
                                      preferred_element_type=jnp.float32)
        mu_out_ref[_Z:2 * _Z, :] = jnp.transpose(mus_ref[...], (1, 0))

        cov_c_t = jnp.dot(cc_tab_ref[...], onehot_t,
                          preferred_element_type=jnp.float32)
        w = lax.bitcast_convert_type(rows_buf[pl.ds((step % 2) * bb, bb)],
                                     jnp.uint32)
        lo = lax.bitcast_convert_type((w & 0xFFFF).astype(jnp.uint16),
                                      jnp.bfloat16)
        hi = lax.bitcast_convert_type((w >> 16).astype(jnp.uint16),
                                      jnp.bfloat16)
        covs_rows = jnp.concatenate([lo, hi], axis=1)
        ls = jnp.reshape(
            jnp.dot(covs_rows, s_ref[...].astype(jnp.bfloat16),
                    preferred_element_type=jnp.float32),
            (bb, _Z, _Z))
        cov_s = lax.dot_general(ls, ls, (((2,), (2,)), ((0,), (0,))),
                                preferred_element_type=jnp.float32)
        cov_s_t = jnp.transpose(jnp.reshape(cov_s, (bb, _Z * _Z)), (1, 0))
        zero = jnp.zeros((_Z, _Z, bb), jnp.float32)
        cov_out_ref[0:_Z, 0:_Z, :] = jnp.reshape(cov_c_t, (_Z, _Z, bb))
        cov_out_ref[0:_Z, _Z:2 * _Z, :] = zero
        cov_out_ref[_Z:2 * _Z, 0:_Z, :] = zero
        cov_out_ref[_Z:2 * _Z, _Z:2 * _Z, :] = jnp.reshape(cov_s_t,
                                                           (_Z, _Z, bb))

    return pl.pallas_call(
        body,
        grid=(grid,),
        in_specs=[
            pl.BlockSpec((1, bb), lambda i: (0, i)),
            pl.BlockSpec((_Z, _NE), lambda i: (0, 0)),
            pl.BlockSpec((_NE, _NT), lambda i: (0, 0)),
            pl.BlockSpec(memory_space=pltpu.SMEM),
            pl.BlockSpec(memory_space=pltpu.SMEM),
            pl.BlockSpec((bb, _Z), lambda i: (i, 0)),
            pl.BlockSpec(memory_space=pl.ANY),
            pl.BlockSpec((_NT, _Z * _Z), lambda i: (0, 0)),
        ],
        out_specs=[
            pl.BlockSpec((2 * _Z, bb), lambda i: (0, i)),
            pl.BlockSpec((2 * _Z, 2 * _Z, bb), lambda i: (0, 0, i)),
        ],
        out_shape=[
            jax.ShapeDtypeStruct((2 * _Z, _B), jnp.float32),
            jax.ShapeDtypeStruct((2 * _Z, 2 * _Z, _B), jnp.float32),
        ],
        scratch_shapes=[
            pltpu.VMEM((_Z * _Z, _NE), jnp.float32),
            pltpu.VMEM((2 * bb, _NT // 2), jnp.float32),
            pltpu.SemaphoreType.DMA,
            pltpu.SemaphoreType.DMA,
        ],
        interpret=interpret,
    )(e_row, muc_t, cov_causal, y_flat, e_flat, mu_s_rows, cov_rm, scatter)


def kernel(y, e, mu_causal, cov_causal, mu_spurious, cov_spurious):
    y_flat = y.reshape(_B).astype(jnp.int32)
    e_flat = e.reshape(_B).astype(jnp.int32)
    scatter = jnp.asarray(_SCATTER_NP)

    cov_sp_t = jnp.transpose(cov_spurious, (1, 2, 0)).reshape(_NE * _NT, _NC)

    cov_rm = _tc_transpose_table(cov_sp_t)
    mu_s_rows = _sc_gather_mu(y_flat, e_flat,
                              mu_spurious.reshape(_NC * _NE, _Z))

    mu_t, cov_t = _tc_assemble(e_flat.reshape(1, _B), mu_causal.T, cov_causal,
                               y_flat, e_flat, mu_s_rows, cov_rm, scatter)
    return (jnp.transpose(mu_t, (1, 0)), jnp.transpose(cov_t, (2, 0, 1)))


# speedup vs baseline: 1.3872x; 1.3707x over previous
import functools

import numpy as np
import jax
import jax.numpy as jnp
from jax import lax
from jax.experimental import pallas as pl
from jax.experimental.pallas import tpu as pltpu
from jax.experimental.pallas import tpu_sc as plsc

_Z = 32
_NT = _Z * (_Z + 1) // 2
_NTH = _NT // 2
_B = 4096
_NE = 64
_NC = 1000


def _build_scatter_matrix():
    s = np.zeros((_NT, _Z * _Z), dtype=np.float32)
    t = 0
    for i in range(_Z):
        for j in range(i + 1):
            s[t, i * _Z + j] = 1.0
            t += 1
    return s


_SCATTER_NP = _build_scatter_matrix()


def _pack_halves(x):
    h = x.shape[1] // 2
    u = lax.bitcast_convert_type(x.astype(jnp.bfloat16),
                                 jnp.uint16).astype(jnp.uint32)
    return lax.bitcast_convert_type(u[:, :h] | (u[:, h:] << 16), jnp.float32)


def _unpack_halves(w):
    u = lax.bitcast_convert_type(w, jnp.uint32)
    lo = lax.bitcast_convert_type((u & 0xFFFF).astype(jnp.uint16),
                                  jnp.bfloat16)
    hi = lax.bitcast_convert_type((u >> 16).astype(jnp.uint16), jnp.bfloat16)
    return jnp.concatenate([lo, hi], axis=1)


def _tc_transpose_tables(cov_sp_t2d, mu_sp_t2d):
    def body(in_c_ref, in_m_ref, t0_ref, t1_ref, t2_ref):
        cov_w = _pack_halves(jnp.transpose(in_c_ref[...], (1, 0)))
        mu_w = _pack_halves(jnp.transpose(in_m_ref[...], (1, 0)))
        t0_ref[...] = cov_w[:, 0:128]
        t1_ref[...] = cov_w[:, 128:256]
        t2_ref[...] = jnp.concatenate(
            [cov_w[:, 256:_NTH], mu_w,
             jnp.zeros((_NC, 128 - (_NTH - 256) - _Z // 2), jnp.float32)],
            axis=1)

    return pl.pallas_call(
        body,
        grid=(_NE,),
        in_specs=[
            pl.BlockSpec((_NT, _NC), lambda e: (e, 0)),
            pl.BlockSpec((_Z, _NC), lambda e: (e, 0)),
        ],
        out_specs=[
            pl.BlockSpec((_NC, 128), lambda e: (e, 0)),
            pl.BlockSpec((_NC, 128), lambda e: (e, 0)),
            pl.BlockSpec((_NC, 128), lambda e: (e, 0)),
        ],
        out_shape=[
            jax.ShapeDtypeStruct((_NE * _NC, 128), jnp.float32),
            jax.ShapeDtypeStruct((_NE * _NC, 128), jnp.float32),
            jax.ShapeDtypeStruct((_NE * _NC, 128), jnp.float32),
        ],
    )(cov_sp_t2d, mu_sp_t2d)


def _sc_gather_rows(y_flat, e_flat, t0, t1, t2):
    info = plsc.get_sparse_core_info()
    num_cores, num_subcores = info.num_cores, info.num_subcores
    nw = num_cores * num_subcores
    bpw = _B // nw
    lanes = info.num_lanes

    mesh = plsc.VectorSubcoreMesh(core_axis_name="c", subcore_axis_name="s")

    @functools.partial(
        pl.kernel,
        out_type=(
            jax.ShapeDtypeStruct((_B, 128), jnp.float32),
            jax.ShapeDtypeStruct((_B, 128), jnp.float32),
            jax.ShapeDtypeStruct((_B, 128), jnp.float32),
        ),
        mesh=mesh,
        scratch_types=[
            pltpu.VMEM((bpw,), jnp.int32),
            pltpu.VMEM((bpw,), jnp.int32),
            pltpu.VMEM((bpw,), jnp.int32),
            pltpu.VMEM((bpw, 128), jnp.float32),
            pltpu.VMEM((bpw, 128), jnp.float32),
            pltpu.VMEM((bpw, 128), jnp.float32),
            pltpu.SemaphoreType.DMA,
            pltpu.SemaphoreType.DMA,
            pltpu.SemaphoreType.DMA,
        ],
        compiler_params=pltpu.CompilerParams(use_tc_tiling_on_sc=False),
    )
    def gather_kernel(y_hbm, e_hbm, t0_hbm, t1_hbm, t2_hbm,
                      o0_hbm, o1_hbm, o2_hbm,
                      y_v, e_v, idx_v, b0, b1, b2, s0, s1, s2):
        wid = lax.axis_index("s") * num_cores + lax.axis_index("c")
        base = wid * bpw
        pltpu.sync_copy(y_hbm.at[pl.ds(base, bpw)], y_v)
        pltpu.sync_copy(e_hbm.at[pl.ds(base, bpw)], e_v)
        for i in range(bpw // lanes):
            sl = pl.ds(i * lanes, lanes)
            idx_v[sl] = e_v[sl] * _NC + y_v[sl]
        c0 = pltpu.async_copy(t0_hbm.at[idx_v], b0, s0)
        c1 = pltpu.async_copy(t1_hbm.at[idx_v], b1, s1)
        c2 = pltpu.async_copy(t2_hbm.at[idx_v], b2, s2)
        c0.wait()
        c1.wait()
        c2.wait()
        pltpu.sync_copy(b0, o0_hbm.at[pl.ds(base, bpw)])
        pltpu.sync_copy(b1, o1_hbm.at[pl.ds(base, bpw)])
        pltpu.sync_copy(b2, o2_hbm.at[pl.ds(base, bpw)])

    return gather_kernel(y_flat, e_flat, t0, t1, t2)


def _tc_assemble(e_row, muc_t, cov_causal, r0, r1, r2, scatter,
                 interpret=False):
    bb = 256
    grid = _B // bb

    def body(e_ref, muct_ref, covc_ref, r0_ref, r1_ref, r2_ref, s_ref,
             mu_out_ref, cov_out_ref, cc_tab_ref):
        step = pl.program_id(0)

        @pl.when(step == 0)
        def _():
            lc = jnp.reshape(
                jnp.dot(covc_ref[...], s_ref[...],
                        preferred_element_type=jnp.float32),
                (_NE, _Z, _Z))
            cc = lax.dot_general(lc, lc, (((2,), (2,)), ((0,), (0,))),
                                 preferred_element_type=jnp.float32)
            cc_tab_ref[...] = jnp.transpose(
                jnp.reshape(cc, (_NE, _Z * _Z)), (1, 0))

        onehot_t = (lax.broadcast_in_dim(e_ref[...], (_NE, bb), (0, 1))
                    == lax.broadcasted_iota(jnp.int32, (_NE, bb), 0)
                    ).astype(jnp.float32)

        cov_w = jnp.concatenate(
            [r0_ref[...], r1_ref[...], r2_ref[:, 0:_NTH - 256]], axis=1)
        covs_rows = _unpack_halves(cov_w)
        mu_s = _unpack_halves(r2_ref[:, _NTH - 256:_NTH - 256 + _Z // 2])

        mu_out_ref[0:_Z, :] = jnp.dot(muct_ref[...], onehot_t,
                                      preferred_element_type=jnp.float32)
        mu_out_ref[_Z:2 * _Z, :] = jnp.transpose(
            mu_s.astype(jnp.float32), (1, 0))

        cov_c_t = jnp.dot(cc_tab_ref[...], onehot_t,
                          preferred_element_type=jnp.float32)
        ls = jnp.reshape(
            jnp.dot(covs_rows, s_ref[...].astype(jnp.bfloat16),
                    preferred_element_type=jnp.float32),
            (bb, _Z, _Z))
        cov_s = lax.dot_general(ls, ls, (((2,), (2,)), ((0,), (0,))),
                                preferred_element_type=jnp.float32)
        cov_s_t = jnp.transpose(jnp.reshape(cov_s, (bb, _Z * _Z)), (1, 0))
        zero = jnp.zeros((_Z, _Z, bb), jnp.float32)
        cov_out_ref[0:_Z, 0:_Z, :] = jnp.reshape(cov_c_t, (_Z, _Z, bb))
        cov_out_ref[0:_Z, _Z:2 * _Z, :] = zero
        cov_out_ref[_Z:2 * _Z, 0:_Z, :] = zero
        cov_out_ref[_Z:2 * _Z, _Z:2 * _Z, :] = jnp.reshape(cov_s_t,
                                                           (_Z, _Z, bb))

    return pl.pallas_call(
        body,
        grid=(grid,),
        in_specs=[
            pl.BlockSpec((1, bb), lambda i: (0, i)),
            pl.BlockSpec((_Z, _NE), lambda i: (0, 0)),
            pl.BlockSpec((_NE, _NT), lambda i: (0, 0)),
            pl.BlockSpec((bb, 128), lambda i: (i, 0)),
            pl.BlockSpec((bb, 128), lambda i: (i, 0)),
            pl.BlockSpec((bb, 128), lambda i: (i, 0)),
            pl.BlockSpec((_NT, _Z * _Z), lambda i: (0, 0)),
        ],
        out_specs=[
            pl.BlockSpec((2 * _Z, bb), lambda i: (0, i)),
            pl.BlockSpec((2 * _Z, 2 * _Z, bb), lambda i: (0, 0, i)),
        ],
        out_shape=[
            jax.ShapeDtypeStruct((2 * _Z, _B), jnp.float32),
            jax.ShapeDtypeStruct((2 * _Z, 2 * _Z, _B), jnp.float32),
        ],
        scratch_shapes=[pltpu.VMEM((_Z * _Z, _NE), jnp.float32)],
        interpret=interpret,
    )(e_row, muc_t, cov_causal, r0, r1, r2, scatter)


def kernel(y, e, mu_causal, cov_causal, mu_spurious, cov_spurious):
    y_flat = y.reshape(_B).astype(jnp.int32)
    e_flat = e.reshape(_B).astype(jnp.int32)
    scatter = jnp.asarray(_SCATTER_NP)

    cov_sp_t = jnp.transpose(cov_spurious, (1, 2, 0)).reshape(_NE * _NT, _NC)
    mu_sp_t = jnp.transpose(mu_spurious, (1, 2, 0)).reshape(_NE * _Z, _NC)

    t0, t1, t2 = _tc_transpose_tables(cov_sp_t, mu_sp_t)
    r0, r1, r2 = _sc_gather_rows(y_flat, e_flat, t0, t1, t2)

    mu_t, cov_t = _tc_assemble(e_flat.reshape(1, _B), mu_causal.T, cov_causal,
                               r0, r1, r2, scatter)
    return (jnp.transpose(mu_t, (1, 0)), jnp.transpose(cov_t, (2, 0, 1)))
